# Initial kernel scaffold; baseline (speedup 1.0000x reference)
import jax
import jax.numpy as jnp
from jax.experimental import pallas as pl


def kernel(x, W_enc, b_enc, W_dec, b_dec):
    raise NotImplementedError("write your pallas kernel here")


# speedup vs baseline: 6.7419x; 6.7419x over previous
import functools

import jax
import jax.numpy as jnp
from jax.experimental import pallas as pl
from jax.experimental.pallas import tpu as pltpu

D_MODEL = 768
D_LATENT = 12288
TOPK = 20
N_TOKENS = 8192

BR_ENC = 256
BC_ENC = 2048
BR_MSK = 128
BR_DEC = 256
BK_DEC = 2048


def _enc_body(x_ref, we_ref, be_ref, lat_ref):
    acc = jnp.dot(x_ref[...].astype(jnp.bfloat16),
                  we_ref[...].astype(jnp.bfloat16),
                  preferred_element_type=jnp.float32)
    lat_ref[...] = acc + be_ref[...]


def _mask_body(lat_ref, out_ref):
    lat = lat_ref[...]
    work = lat
    thresh = None
    for _ in range(TOPK):
        thresh = jnp.max(work, axis=1, keepdims=True)
        work = jnp.where(work >= thresh, -jnp.inf, work)
    out_ref[...] = jnp.where(lat >= thresh, lat, 0.0)


def _dec_body(s_ref, wd_ref, bd_ref, out_ref):
    j = pl.program_id(1)

    @pl.when(j == 0)
    def _():
        out_ref[...] = jnp.broadcast_to(bd_ref[...], out_ref.shape)

    out_ref[...] += jnp.dot(s_ref[...].astype(jnp.bfloat16),
                            wd_ref[...].astype(jnp.bfloat16),
                            preferred_element_type=jnp.float32)


@jax.jit
def kernel(x, W_enc, b_enc, W_dec, b_dec):
    be2 = b_enc.reshape(1, D_LATENT)
    bd2 = b_dec.reshape(1, D_MODEL)

    latents = pl.pallas_call(
        _enc_body,
        grid=(N_TOKENS // BR_ENC, D_LATENT // BC_ENC),
        in_specs=[
            pl.BlockSpec((BR_ENC, D_MODEL), lambda i, j: (i, 0)),
            pl.BlockSpec((D_MODEL, BC_ENC), lambda i, j: (0, j)),
            pl.BlockSpec((1, BC_ENC), lambda i, j: (0, j)),
        ],
        out_specs=pl.BlockSpec((BR_ENC, BC_ENC), lambda i, j: (i, j)),
        out_shape=jax.ShapeDtypeStruct((N_TOKENS, D_LATENT), jnp.float32),
        compiler_params=pltpu.CompilerParams(
            dimension_semantics=("parallel", "parallel")),
    )(x, W_enc, be2)

    masked = pl.pallas_call(
        _mask_body,
        grid=(N_TOKENS // BR_MSK,),
        in_specs=[pl.BlockSpec((BR_MSK, D_LATENT), lambda i: (i, 0))],
        out_specs=pl.BlockSpec((BR_MSK, D_LATENT), lambda i: (i, 0)),
        out_shape=jax.ShapeDtypeStruct((N_TOKENS, D_LATENT), jnp.float32),
        compiler_params=pltpu.CompilerParams(
            dimension_semantics=("parallel",)),
    )(latents)

    recons = pl.pallas_call(
        _dec_body,
        grid=(N_TOKENS // BR_DEC, D_LATENT // BK_DEC),
        in_specs=[
            pl.BlockSpec((BR_DEC, BK_DEC), lambda i, j: (i, j)),
            pl.BlockSpec((BK_DEC, D_MODEL), lambda i, j: (j, 0)),
            pl.BlockSpec((1, D_MODEL), lambda i, j: (0, 0)),
        ],
        out_specs=pl.BlockSpec((BR_DEC, D_MODEL), lambda i, j: (i, 0)),
        out_shape=jax.ShapeDtypeStruct((N_TOKENS, D_MODEL), jnp.float32),
        compiler_params=pltpu.CompilerParams(
            dimension_semantics=("parallel", "arbitrary")),
    )(masked, W_dec, bd2)

    return recons
